# baseline (speedup 1.0000x reference)
import functools

import jax
import jax.numpy as jnp
from jax import lax
from jax.experimental import pallas as pl
from jax.experimental.pallas import tpu as pltpu
from jax.experimental.pallas import tpu_sc as plsc

_D = 64
_T = 200
_B = 4096
_V = 1000000
_SCALE = 8.0

_info = plsc.get_sparse_core_info()
_NC, _NS, _L = _info.num_cores, _info.num_subcores, _info.num_lanes
_NW = _NC * _NS
_SEQ_PER_W = _B // _NW
_CHUNK_A = 128
_CHUNK_B = _T - _CHUNK_A
_DP = 128

_RCH = 160
_NRC = _V // _RCH


@functools.partial(
    pl.kernel,
    mesh=plsc.VectorSubcoreMesh(core_axis_name="c", subcore_axis_name="s"),
    out_type=jax.ShapeDtypeStruct((_V, _DP), jnp.float32),
    scratch_types=[
        pltpu.VMEM((_RCH, _D), jnp.float32),
        pltpu.VMEM((_RCH, _D), jnp.float32),
        pltpu.VMEM((_RCH, _DP), jnp.float32),
        pltpu.VMEM((_RCH, _DP), jnp.float32),
        pltpu.SemaphoreType.DMA,
        pltpu.SemaphoreType.DMA,
        pltpu.SemaphoreType.DMA,
        pltpu.SemaphoreType.DMA,
    ],
)
def _repack(tok_hbm, out_hbm, a0, a1, b0, b1, gi0, gi1, so0, so1):
    ain = (a0, a1)
    bout = (b0, b1)
    gsem = (gi0, gi1)
    ssem = (so0, so1)
    wid = lax.axis_index("s") * _NC + lax.axis_index("c")

    def issue_in(b, cid):
        pltpu.async_copy(
            tok_hbm.at[pl.ds(cid * _RCH, _RCH)], ain[b], gsem[b])

    def wait_in(b):
        pltpu.make_async_copy(
            tok_hbm.at[pl.ds(0, _RCH)], ain[b], gsem[b]).wait()

    def issue_out(b, cid):
        pltpu.async_copy(
            bout[b], out_hbm.at[pl.ds(cid * _RCH, _RCH)], ssem[b])

    def wait_out(b):
        pltpu.make_async_copy(
            bout[b], out_hbm.at[pl.ds(0, _RCH)], ssem[b]).wait()

    for b in range(2):
        issue_in(b, wid + b * _NW)

    def round_body(k, carry):
        for b in range(2):
            cid = wid + (2 * k + b) * _NW
            live = cid < _NRC

            @pl.when(live)
            def _():
                wait_in(b)

            @pl.when(live & (k > 0))
            def _():
                wait_out(b)

            ain_b, bout_b = ain[b], bout[b]

            @plsc.parallel_loop(0, _RCH, unroll=8)
            def _(r):
                for j in range(_D // _L):
                    sl = pl.ds(j * _L, _L)
                    bout_b[r, sl] = ain_b[r, sl]

            @pl.when(cid + 2 * _NW < _NRC)
            def _():
                issue_in(b, cid + 2 * _NW)

            @pl.when(live)
            def _():
                issue_out(b, cid)
        return carry

    lax.fori_loop(0, 98, round_body, 0)
    for b in range(2):
        wait_out(b)


@functools.partial(
    pl.kernel,
    mesh=plsc.VectorSubcoreMesh(core_axis_name="c", subcore_axis_name="s"),
    compiler_params=pltpu.CompilerParams(use_tc_tiling_on_sc=False),
    out_type=jax.ShapeDtypeStruct((_B * _T, _DP), jnp.float32),
    scratch_types=[
        pltpu.VMEM((_T, _D), jnp.float32),
        pltpu.VMEM((_SEQ_PER_W * _T,), jnp.int32),
        pltpu.VMEM((_T, _DP), jnp.float32),
        pltpu.VMEM((_T, _DP), jnp.float32),
        pltpu.VMEM((_T, _D), jnp.float32),
        pltpu.VMEM((_T, _D), jnp.float32),
        pltpu.SemaphoreType.DMA,
        pltpu.SemaphoreType.DMA,
        pltpu.SemaphoreType.DMA,
        pltpu.SemaphoreType.DMA,
    ],
)
def _embed(x_hbm, tok_hbm, pos_hbm, out_hbm,
           pos_v, idx_v, rin0, rin1, rout0, rout1, gs0, gs1, ss0, ss1):
    rin = (rin0, rin1)
    rout = (rout0, rout1)
    gsem = (gs0, gs1)
    ssem = (ss0, ss1)
    wid = lax.axis_index("s") * _NC + lax.axis_index("c")
    seq0 = wid * _SEQ_PER_W

    pltpu.sync_copy(pos_hbm.at[pl.ds(0, _T)], pos_v)
    pltpu.sync_copy(x_hbm.at[pl.ds(seq0 * _T, _SEQ_PER_W * _T)], idx_v)

    def issue_gather(b, s):
        off = s * _T
        pltpu.async_copy(
            tok_hbm.at[idx_v.at[pl.ds(off, _CHUNK_A)]],
            rin[b].at[pl.ds(0, _CHUNK_A)], gsem[b])
        pltpu.async_copy(
            tok_hbm.at[idx_v.at[pl.ds(off + _CHUNK_A, _CHUNK_B)]],
            rin[b].at[pl.ds(_CHUNK_A, _CHUNK_B)], gsem[b])

    def wait_gather(b):
        pltpu.make_async_copy(tok_hbm.at[pl.ds(0, _T)], rin[b], gsem[b]).wait()

    def issue_store(b, s):
        pltpu.async_copy(
            rout[b],
            out_hbm.at[pl.ds((seq0 + s) * _T, _T), pl.ds(0, _D)], ssem[b])

    def wait_store(b):
        pltpu.make_async_copy(
            rout[b], out_hbm.at[pl.ds(0, _T), pl.ds(0, _D)], ssem[b]).wait()

    for b in range(2):
        issue_gather(b, b)

    def round_body(k, carry):
        for b in range(2):
            s = k * 2 + b
            wait_gather(b)

            @pl.when(k > 0)
            def _():
                wait_store(b)

            rin_b, rout_b = rin[b], rout[b]

            @plsc.parallel_loop(0, _T, unroll=8)
            def _(r):
                for j in range(_D // _L):
                    sl = pl.ds(j * _L, _L)
                    rout_b[r, sl] = rin_b[r, sl] * _SCALE + pos_v[r, sl]

            @pl.when(s + 2 < _SEQ_PER_W)
            def _():
                issue_gather(b, s + 2)

            issue_store(b, s)
        return carry

    lax.fori_loop(0, _SEQ_PER_W // 2, round_body, 0)
    for b in range(2):
        wait_store(b)


def kernel(x, token_table, pos_table):
    b, t = x.shape
    tab = _repack(token_table)
    out = _embed(x.reshape(b * t).astype(jnp.int32), tab, pos_table)
    return out[:, :_D].reshape(b, t, _D)


# speedup vs baseline: 1.1707x; 1.1707x over previous
import functools

import jax
import jax.numpy as jnp
from jax import lax
from jax.experimental import pallas as pl
from jax.experimental.pallas import tpu as pltpu
from jax.experimental.pallas import tpu_sc as plsc

_D = 64
_T = 200
_B = 4096
_SCALE = 8.0

_info = plsc.get_sparse_core_info()
_NC, _NS, _L = _info.num_cores, _info.num_subcores, _info.num_lanes
_NW = _NC * _NS
_SEQ_PER_W = _B // _NW
_CHUNK_A = 128
_CHUNK_B = _T - _CHUNK_A
_DP = 128


@functools.partial(
    pl.kernel,
    mesh=plsc.VectorSubcoreMesh(core_axis_name="c", subcore_axis_name="s"),
    compiler_params=pltpu.CompilerParams(use_tc_tiling_on_sc=False),
    out_type=jax.ShapeDtypeStruct((_B * _T, _DP), jnp.float32),
    scratch_types=[
        pltpu.VMEM((_T, _D), jnp.float32),
        pltpu.VMEM((_SEQ_PER_W * _T,), jnp.int32),
        pltpu.VMEM((_T, _D), jnp.float32),
        pltpu.VMEM((_T, _D), jnp.float32),
        pltpu.VMEM((_T, _D), jnp.float32),
        pltpu.VMEM((_T, _D), jnp.float32),
        pltpu.SemaphoreType.DMA,
        pltpu.SemaphoreType.DMA,
        pltpu.SemaphoreType.DMA,
        pltpu.SemaphoreType.DMA,
    ],
)
def _embed(x_hbm, tok_hbm, pos_hbm, out_hbm,
           pos_v, idx_v, rin0, rin1, rout0, rout1, gs0, gs1, ss0, ss1):
    rin = (rin0, rin1)
    rout = (rout0, rout1)
    gsem = (gs0, gs1)
    ssem = (ss0, ss1)
    wid = lax.axis_index("s") * _NC + lax.axis_index("c")
    seq0 = wid * _SEQ_PER_W

    pltpu.sync_copy(pos_hbm.at[pl.ds(0, _T)], pos_v)
    pltpu.sync_copy(x_hbm.at[pl.ds(seq0 * _T, _SEQ_PER_W * _T)], idx_v)

    def issue_gather(b, s):
        off = s * _T
        pltpu.async_copy(
            tok_hbm.at[idx_v.at[pl.ds(off, _CHUNK_A)]],
            rin[b].at[pl.ds(0, _CHUNK_A)], gsem[b])
        pltpu.async_copy(
            tok_hbm.at[idx_v.at[pl.ds(off + _CHUNK_A, _CHUNK_B)]],
            rin[b].at[pl.ds(_CHUNK_A, _CHUNK_B)], gsem[b])

    def wait_gather(b):
        pltpu.make_async_copy(tok_hbm.at[pl.ds(0, _T)], rin[b], gsem[b]).wait()

    def issue_store(b, s):
        pltpu.async_copy(
            rout[b],
            out_hbm.at[pl.ds((seq0 + s) * _T, _T), pl.ds(0, _D)], ssem[b])

    def wait_store(b):
        pltpu.make_async_copy(
            rout[b], out_hbm.at[pl.ds(0, _T), pl.ds(0, _D)], ssem[b]).wait()

    for b in range(2):
        issue_gather(b, b)

    def round_body(k, carry):
        for b in range(2):
            s = k * 2 + b
            wait_gather(b)

            @pl.when(k > 0)
            def _():
                wait_store(b)

            rin_b, rout_b = rin[b], rout[b]

            @plsc.parallel_loop(0, _T, unroll=8)
            def _(r):
                for j in range(_D // _L):
                    sl = pl.ds(j * _L, _L)
                    rout_b[r, sl] = rin_b[r, sl] * _SCALE + pos_v[r, sl]

            @pl.when(s + 2 < _SEQ_PER_W)
            def _():
                issue_gather(b, s + 2)

            issue_store(b, s)
        return carry

    lax.fori_loop(0, _SEQ_PER_W // 2, round_body, 0)
    for b in range(2):
        wait_store(b)


def kernel(x, token_table, pos_table):
    b, t = x.shape
    out = _embed(x.reshape(b * t).astype(jnp.int32), token_table, pos_table)
    return out[:, :_D].reshape(b, t, _D)
